# baseline (speedup 1.0000x reference)
import functools

import jax
import jax.numpy as jnp
from jax import lax
from jax.experimental import pallas as pl
from jax.experimental.pallas import tpu as pltpu


def _process(s_ref, g_ref, go_ref, ng_ref, j, *, r, n, k):
    band = s_ref[:, pl.ds(j * r, r)]
    eye = (
        lax.broadcasted_iota(jnp.int32, (r, r), 0)
        == lax.broadcasted_iota(jnp.int32, (r, r), 1)
    )
    s_ref[:, pl.ds(j * r, r)] = jnp.where(eye, 0.0, band)

    scores = s_ref[...]
    neg = jnp.float32(-jnp.inf)
    m = jnp.max(scores, axis=1, keepdims=True)
    tot = m
    for _ in range(k - 1):
        m = jnp.max(jnp.where(scores < m, scores, neg), axis=1, keepdims=True)
        tot = tot + m

    vals = jnp.where(scores >= m, scores, 0.0)
    s = jnp.where(tot > 0, tot, 1.0)
    ng = vals / s
    ng_ref[...] = ng

    gb = g_ref[...]
    gs = jnp.sum(gb, axis=1, keepdims=True)
    gs = jnp.where(gs > 0, gs, 1.0)
    s2 = jnp.where(gs > 0, 1.0, gs) + jnp.where(tot > 0, 1.0, tot)
    s2 = jnp.where(s2 > 0, s2, 1.0)
    go_ref[...] = (gb / gs + ng) / s2


def _main_kernel(g_ref, hn_ref, hb_ref, go_ref, ng_ref, s0_ref, s1_ref,
                 *, r, n, k, nb):
    i = pl.program_id(0)
    even = (i % 2) == 0
    scores = lax.dot_general(
        hb_ref[...], hn_ref[...], (((1,), (1,)), ((), ())),
        preferred_element_type=jnp.float32,
    )

    @pl.when((i < nb) & even)
    def _():
        s0_ref[...] = scores

    @pl.when((i < nb) & jnp.logical_not(even))
    def _():
        s1_ref[...] = scores

    proc = functools.partial(_process, g_ref=g_ref, go_ref=go_ref,
                             ng_ref=ng_ref, j=i - 1, r=r, n=n, k=k)

    @pl.when((i > 0) & even)
    def _():
        proc(s1_ref)

    @pl.when((i > 0) & jnp.logical_not(even))
    def _():
        proc(s0_ref)


def kernel(g, h):
    n, d = h.shape
    k = 5
    r = min(128, n)
    nb = n // r

    nrm = jnp.linalg.norm(h, axis=1, keepdims=True)
    hn = h / jnp.clip(nrm, 1e-12)

    body = functools.partial(_main_kernel, r=r, n=n, k=k, nb=nb)
    go, ng = pl.pallas_call(
        body,
        grid=(nb + 1,),
        in_specs=[
            pl.BlockSpec((r, n), lambda i: (jnp.maximum(i - 1, 0), 0)),
            pl.BlockSpec((n, d), lambda i: (0, 0)),
            pl.BlockSpec((r, d), lambda i: (jnp.minimum(i, nb - 1), 0)),
        ],
        out_specs=[
            pl.BlockSpec((r, n), lambda i: (jnp.maximum(i - 1, 0), 0)),
            pl.BlockSpec((r, n), lambda i: (jnp.maximum(i - 1, 0), 0)),
        ],
        out_shape=[
            jax.ShapeDtypeStruct((n, n), jnp.float32),
            jax.ShapeDtypeStruct((n, n), jnp.float32),
        ],
        scratch_shapes=[
            pltpu.VMEM((r, n), jnp.float32),
            pltpu.VMEM((r, n), jnp.float32),
        ],
    )(g, hn, hn)
    return (go, ng)


# speedup vs baseline: 1.1194x; 1.1194x over previous
import functools

import jax
import jax.numpy as jnp
from jax import lax
from jax.experimental import pallas as pl
from jax.experimental.pallas import tpu as pltpu

_LANES = 128


def _process(s_ref, prev, g_ref, go_ref, ng_ref, j, *, r, n, k):
    band = s_ref[prev, :, pl.ds(j * r, r)]
    eye = (
        lax.broadcasted_iota(jnp.int32, (r, r), 0)
        == lax.broadcasted_iota(jnp.int32, (r, r), 1)
    )
    s_ref[prev, :, pl.ds(j * r, r)] = jnp.where(eye, 0.0, band)

    neg = jnp.float32(-jnp.inf)
    t = [jnp.full((r, _LANES), neg, jnp.float32) for _ in range(k)]
    for c in range(n // _LANES):
        v = s_ref[prev, :, pl.ds(c * _LANES, _LANES)]
        for lvl in range(k):
            hi = jnp.maximum(t[lvl], v)
            v = jnp.minimum(t[lvl], v)
            t[lvl] = hi

    cand = jnp.concatenate(t, axis=1)
    m = jnp.max(cand, axis=1, keepdims=True)
    tot = m
    for _ in range(k - 1):
        m = jnp.max(jnp.where(cand < m, cand, neg), axis=1, keepdims=True)
        tot = tot + m

    scores = s_ref[prev]
    s = jnp.where(tot > 0, tot, 1.0)
    ng = jnp.where(scores >= m, scores, 0.0) / s
    ng_ref[...] = ng

    gb = g_ref[...]
    gs = jnp.sum(gb, axis=1, keepdims=True)
    gs = jnp.where(gs > 0, gs, 1.0)
    s2 = jnp.where(gs > 0, 1.0, gs) + jnp.where(tot > 0, 1.0, tot)
    s2 = jnp.where(s2 > 0, s2, 1.0)
    go_ref[...] = (gb / gs + ng) / s2


def _main_kernel(g_ref, hn_ref, hb_ref, go_ref, ng_ref, s_ref,
                 *, r, n, k, nb):
    i = pl.program_id(0)
    cur = lax.rem(i, 2)
    prev = 1 - cur

    @pl.when(i > 0)
    def _():
        _process(s_ref, prev, g_ref, go_ref, ng_ref, i - 1, r=r, n=n, k=k)

    scores = lax.dot_general(
        hb_ref[...], hn_ref[...], (((1,), (1,)), ((), ())),
        preferred_element_type=jnp.float32,
    )

    @pl.when(i < nb)
    def _():
        s_ref[cur] = scores


def kernel(g, h):
    n, d = h.shape
    k = 5
    r = min(128, n)
    nb = n // r

    nrm = jnp.linalg.norm(h, axis=1, keepdims=True)
    hn = h / jnp.clip(nrm, 1e-12)

    body = functools.partial(_main_kernel, r=r, n=n, k=k, nb=nb)
    go, ng = pl.pallas_call(
        body,
        grid=(nb + 1,),
        in_specs=[
            pl.BlockSpec((r, n), lambda i: (jnp.maximum(i - 1, 0), 0)),
            pl.BlockSpec((n, d), lambda i: (0, 0)),
            pl.BlockSpec((r, d), lambda i: (jnp.minimum(i, nb - 1), 0)),
        ],
        out_specs=[
            pl.BlockSpec((r, n), lambda i: (jnp.maximum(i - 1, 0), 0)),
            pl.BlockSpec((r, n), lambda i: (jnp.maximum(i - 1, 0), 0)),
        ],
        out_shape=[
            jax.ShapeDtypeStruct((n, n), jnp.float32),
            jax.ShapeDtypeStruct((n, n), jnp.float32),
        ],
        scratch_shapes=[
            pltpu.VMEM((2, r, n), jnp.float32),
        ],
    )(g, hn, hn)
    return (go, ng)


# speedup vs baseline: 1.4546x; 1.2995x over previous
import functools

import jax
import jax.numpy as jnp
from jax import lax
from jax.experimental import pallas as pl
from jax.experimental.pallas import tpu as pltpu

_LANES = 128


def _process(s_ref, prev, g_ref, go_ref, ng_ref, j, *, r, n, k):
    band = s_ref[prev, :, pl.ds(j * r, r)]
    eye = (
        lax.broadcasted_iota(jnp.int32, (r, r), 0)
        == lax.broadcasted_iota(jnp.int32, (r, r), 1)
    )
    s_ref[prev, :, pl.ds(j * r, r)] = jnp.where(eye, 0.0, band)

    neg = jnp.float32(-jnp.inf)
    t = [jnp.full((r, _LANES), neg, jnp.float32) for _ in range(k)]
    for c in range(n // _LANES):
        v = s_ref[prev, :, pl.ds(c * _LANES, _LANES)]
        for lvl in range(k):
            hi = jnp.maximum(t[lvl], v)
            v = jnp.minimum(t[lvl], v)
            t[lvl] = hi

    cand = jnp.concatenate(t, axis=1)
    m = jnp.max(cand, axis=1, keepdims=True)
    tot = m
    for _ in range(k - 1):
        m = jnp.max(jnp.where(cand < m, cand, neg), axis=1, keepdims=True)
        tot = tot + m

    scores = s_ref[prev]
    s = jnp.where(tot > 0, tot, 1.0)
    ng = jnp.where(scores >= m, scores, 0.0) / s
    ng_ref[...] = ng

    gb = g_ref[...]
    gs = jnp.sum(gb, axis=1, keepdims=True)
    gs = jnp.where(gs > 0, gs, 1.0)
    s2 = jnp.where(gs > 0, 1.0, gs) + jnp.where(tot > 0, 1.0, tot)
    s2 = jnp.where(s2 > 0, s2, 1.0)
    go_ref[...] = (gb / gs + ng) / s2


def _main_kernel(g_ref, hn_ref, hb_ref, go_ref, ng_ref, s_ref,
                 *, r, n, k, nb):
    i = pl.program_id(0)
    cur = lax.rem(i, 2)
    prev = 1 - cur

    @pl.when(i > 0)
    def _():
        _process(s_ref, prev, g_ref, go_ref, ng_ref, i - 1, r=r, n=n, k=k)

    scores = lax.dot_general(
        hb_ref[...], hn_ref[...], (((1,), (1,)), ((), ())),
        preferred_element_type=jnp.float32,
    )

    @pl.when(i < nb)
    def _():
        s_ref[cur] = scores


def kernel(g, h):
    n, d = h.shape
    k = 5
    r = min(256, n)
    nb = n // r

    nrm = jnp.linalg.norm(h, axis=1, keepdims=True)
    hn = h / jnp.clip(nrm, 1e-12)

    body = functools.partial(_main_kernel, r=r, n=n, k=k, nb=nb)
    go, ng = pl.pallas_call(
        body,
        grid=(nb + 1,),
        in_specs=[
            pl.BlockSpec((r, n), lambda i: (jnp.maximum(i - 1, 0), 0)),
            pl.BlockSpec((n, d), lambda i: (0, 0)),
            pl.BlockSpec((r, d), lambda i: (jnp.minimum(i, nb - 1), 0)),
        ],
        out_specs=[
            pl.BlockSpec((r, n), lambda i: (jnp.maximum(i - 1, 0), 0)),
            pl.BlockSpec((r, n), lambda i: (jnp.maximum(i - 1, 0), 0)),
        ],
        out_shape=[
            jax.ShapeDtypeStruct((n, n), jnp.float32),
            jax.ShapeDtypeStruct((n, n), jnp.float32),
        ],
        scratch_shapes=[
            pltpu.VMEM((2, r, n), jnp.float32),
        ],
    )(g, hn, hn)
    return (go, ng)


# speedup vs baseline: 1.4598x; 1.0035x over previous
import functools

import jax
import jax.numpy as jnp
from jax import lax
from jax.experimental import pallas as pl
from jax.experimental.pallas import tpu as pltpu

_LANES = 128


def _process(s_ref, prev, g_ref, go_ref, ng_ref, j, *, r, n, k):
    band = s_ref[prev, :, pl.ds(j * r, r)]
    eye = (
        lax.broadcasted_iota(jnp.int32, (r, r), 0)
        == lax.broadcasted_iota(jnp.int32, (r, r), 1)
    )
    s_ref[prev, :, pl.ds(j * r, r)] = jnp.where(eye, 0.0, band)

    neg = jnp.float32(-jnp.inf)
    t = [jnp.full((r, _LANES), neg, jnp.float32) for _ in range(k)]
    for c in range(n // _LANES):
        v = s_ref[prev, :, pl.ds(c * _LANES, _LANES)]
        for lvl in range(k):
            hi = jnp.maximum(t[lvl], v)
            if lvl + 1 < k:
                v = jnp.minimum(t[lvl], v)
            t[lvl] = hi

    def masked_row_max(arrs, bound):
        acc = None
        for a in arrs:
            am = jnp.where(a < bound, a, neg)
            acc = am if acc is None else jnp.maximum(acc, am)
        return jnp.max(acc, axis=1, keepdims=True)

    m = jnp.max(t[0], axis=1, keepdims=True)
    tot = m
    for _ in range(k - 1):
        m = masked_row_max(t, m)
        tot = tot + m

    scores = s_ref[prev]
    s = jnp.where(tot > 0, tot, 1.0)
    ng = jnp.where(scores >= m, scores, 0.0) / s
    ng_ref[...] = ng

    gb = g_ref[...]
    gs = jnp.sum(gb, axis=1, keepdims=True)
    gs = jnp.where(gs > 0, gs, 1.0)
    s2 = jnp.where(gs > 0, 1.0, gs) + jnp.where(tot > 0, 1.0, tot)
    s2 = jnp.where(s2 > 0, s2, 1.0)
    go_ref[...] = (gb / gs + ng) / s2


def _main_kernel(g_ref, hn_ref, hb_ref, go_ref, ng_ref, s_ref,
                 *, r, n, k, nb):
    i = pl.program_id(0)
    cur = lax.rem(i, 2)
    prev = 1 - cur

    @pl.when(i > 0)
    def _():
        _process(s_ref, prev, g_ref, go_ref, ng_ref, i - 1, r=r, n=n, k=k)

    scores = lax.dot_general(
        hb_ref[...], hn_ref[...], (((1,), (1,)), ((), ())),
        preferred_element_type=jnp.float32,
    )

    s_ref[cur] = scores


def kernel(g, h):
    n, d = h.shape
    k = 5
    r = min(256, n)
    nb = n // r

    nrm = jnp.linalg.norm(h, axis=1, keepdims=True)
    hn = h / jnp.clip(nrm, 1e-12)

    body = functools.partial(_main_kernel, r=r, n=n, k=k, nb=nb)
    go, ng = pl.pallas_call(
        body,
        grid=(nb + 1,),
        in_specs=[
            pl.BlockSpec((r, n), lambda i: (jnp.maximum(i - 1, 0), 0)),
            pl.BlockSpec((n, d), lambda i: (0, 0)),
            pl.BlockSpec((r, d), lambda i: (jnp.minimum(i, nb - 1), 0)),
        ],
        out_specs=[
            pl.BlockSpec((r, n), lambda i: (jnp.maximum(i - 1, 0), 0)),
            pl.BlockSpec((r, n), lambda i: (jnp.maximum(i - 1, 0), 0)),
        ],
        out_shape=[
            jax.ShapeDtypeStruct((n, n), jnp.float32),
            jax.ShapeDtypeStruct((n, n), jnp.float32),
        ],
        scratch_shapes=[
            pltpu.VMEM((2, r, n), jnp.float32),
        ],
    )(g, hn, hn)
    return (go, ng)
